# baseline (speedup 1.0000x reference)
import functools

import jax
import jax.numpy as jnp
from jax import lax
from jax.experimental import pallas as pl
from jax.experimental.pallas import tpu as pltpu
from jax.experimental.pallas import tpu_sc as plsc


def _sc_gather(idx_flat, table, n, d):
    info = plsc.get_sparse_core_info()
    nc, ns = info.num_cores, info.num_subcores
    nw = nc * ns
    assert n % nw == 0
    per_w = n // nw
    ch = 128
    assert per_w % ch == 0
    n_ch = per_w // ch
    mesh = plsc.VectorSubcoreMesh(core_axis_name="c", subcore_axis_name="s")

    @functools.partial(
        pl.kernel,
        out_type=jax.ShapeDtypeStruct((n, d), jnp.float32),
        mesh=mesh,
        scratch_types=[
            pltpu.VMEM((ch,), jnp.int32),
            pltpu.VMEM((ch, d), jnp.float32),
            pltpu.SemaphoreType.DMA,
        ],
    )
    def gather_kernel(idx_hbm, t_hbm, g_hbm, idx_v, rows, sem):
        wid = lax.axis_index("s") * nc + lax.axis_index("c")
        base = wid * per_w

        @pl.loop(0, n_ch)
        def _(c):
            off = base + c * ch
            pltpu.sync_copy(idx_hbm.at[pl.ds(off, ch)], idx_v)
            pltpu.async_copy(t_hbm.at[idx_v], rows, sem).wait()
            pltpu.sync_copy(rows, g_hbm.at[pl.ds(off, ch)])

    return gather_kernel(idx_flat, table)


def _tc_fuse(g, wt, bias, n, d, dout):
    tb = 2048
    assert n % tb == 0

    def body(g_ref, w_ref, b_ref, out_ref):
        acc = jnp.dot(g_ref[...], w_ref[...], preferred_element_type=jnp.float32)
        out_ref[...] = jnp.maximum(acc + b_ref[...], 0.0)

    return pl.pallas_call(
        body,
        grid=(n // tb,),
        in_specs=[
            pl.BlockSpec((tb, d), lambda i: (i, 0)),
            pl.BlockSpec((d, dout), lambda i: (0, 0)),
            pl.BlockSpec((1, dout), lambda i: (0, 0)),
        ],
        out_specs=pl.BlockSpec((tb, dout), lambda i: (i, 0)),
        out_shape=jax.ShapeDtypeStruct((n, dout), jnp.float32),
    )(g, wt, bias)


def kernel(pad_ques, emb0, emb1, W, b):
    B, L = pad_ques.shape
    n = B * L
    dout = W.shape[0]
    d = emb0.shape[1] + emb1.shape[1]
    table = jnp.concatenate([emb0, emb1], axis=1)
    idx_flat = pad_ques.reshape(n)
    g = _sc_gather(idx_flat, table, n, d)
    out = _tc_fuse(g, W.T, b.reshape(1, dout), n, d, dout)
    return out.reshape(B, L, dout)
